# baseline (speedup 1.0000x reference)
import functools

import jax
import jax.numpy as jnp
from jax import lax
from jax.experimental import pallas as pl
from jax.experimental.pallas import tpu as pltpu
from jax.experimental.pallas import tpu_sc as plsc

N = 10000
NP = 10240
E = 160000
D = 256
DH = 128
B = 80
NTILES = 16
NCH = E // (B * NTILES)
ROWS_PER_TILE = NP // NTILES
NBUF = 3

_mesh = plsc.VectorSubcoreMesh(core_axis_name="c", subcore_axis_name="s")


def _sc_agg_body(x3, src_h, dst_h, w_h, agg3, ws_out,
                 is_0, is_1, is_2, id_0, id_1, id_2, w_0, w_1, w_2,
                 rows_0, rows_1, rows_2,
                 zws_v, acc_sh, ws_sh, gs_0, gs_1, gs_2, ss_0, ss_1, ss_2):
    c = lax.axis_index("c")
    s = lax.axis_index("s")
    bufs = ((is_0, id_0, w_0, rows_0, gs_0, ss_0),
            (is_1, id_1, w_1, rows_1, gs_1, ss_1),
            (is_2, id_2, w_2, rows_2, gs_2, ss_2))

    def zrow(i, carry):
        for k in range(DH // 16):
            rows_0[i, k * 16:(k + 1) * 16] = jnp.zeros((16,), jnp.float32)
        return carry
    lax.fori_loop(0, B, zrow, 0)

    def zws_row(i, carry):
        zws_v[pl.ds(i * 16, 16)] = jnp.zeros((16,), jnp.float32)
        return carry
    lax.fori_loop(0, ROWS_PER_TILE // 16, zws_row, 0)

    base_rows = s * ROWS_PER_TILE
    for kk in range(ROWS_PER_TILE // B):
        pltpu.sync_copy(rows_0, acc_sh.at[pl.ds(base_rows + kk * B, B)])
    pltpu.sync_copy(zws_v, ws_sh.at[pl.ds(base_rows, ROWS_PER_TILE)])

    plsc.subcore_barrier()

    ebase = s * (NCH * B)

    def issue(jj, buf):
        src_v, dst_v, w_v, rows_v, gsem, _ = buf
        base = ebase + jj * B
        pltpu.sync_copy(src_h.at[pl.ds(base, B)], src_v)
        pltpu.sync_copy(dst_h.at[pl.ds(base, B)], dst_v)
        pltpu.sync_copy(w_h.at[pl.ds(base, B)], w_v)
        pltpu.async_copy(x3.at[c].at[src_v], rows_v, gsem)

    issue(0, bufs[0])
    issue(1, bufs[1])

    @pl.loop(0, NCH + (-NCH) % NBUF, step=NBUF)
    def _chunks(j):
        for b in range(NBUF):
            src_v, dst_v, w_v, rows_v, gsem, ssem = bufs[b]
            _, dst_z, _, rows_z, _, ssem_z = bufs[(b + 2) % NBUF]
            jj = j + b

            @pl.when(jj < NCH)
            def _():
                pltpu.make_async_copy(
                    x3.at[c].at[src_v], rows_v, gsem).wait()

                def grp(g, rcarry):
                    wvec = w_v[pl.ds(g * 16, 16)]
                    for r in range(16):
                        wr = wvec[r]
                        row = g * 16 + r
                        for k in range(DH // 16):
                            sl = pl.ds(k * 16, 16)
                            rows_v[row, sl] = rows_v[row, sl] * wr
                    return rcarry
                lax.fori_loop(0, B // 16, grp, 0)

                pltpu.async_copy(rows_v, acc_sh.at[dst_v], ssem, add=True)

                @pl.when((jj & 1) == c)
                def _():
                    pltpu.sync_copy(w_v, ws_sh.at[dst_v], add=True)

                @pl.when(jj >= 1)
                def _():
                    pltpu.make_async_copy(
                        rows_z, acc_sh.at[dst_z], ssem_z).wait()

                @pl.when(jj + 2 < NCH)
                def _():
                    issue(jj + 2, bufs[(b + 2) % NBUF])

    _, dst_l, _, rows_l, _, ssem_l = bufs[(NCH - 1) % NBUF]
    pltpu.make_async_copy(rows_l, acc_sh.at[dst_l], ssem_l).wait()

    plsc.subcore_barrier()

    pltpu.sync_copy(acc_sh.at[pl.ds(base_rows, ROWS_PER_TILE)],
                    agg3.at[c].at[pl.ds(base_rows, ROWS_PER_TILE)])
    pltpu.sync_copy(ws_sh.at[pl.ds(base_rows, ROWS_PER_TILE)],
                    ws_out.at[c].at[pl.ds(base_rows, ROWS_PER_TILE)])


_sc_agg = functools.partial(
    pl.kernel,
    out_type=(jax.ShapeDtypeStruct((2, NP, DH), jnp.float32),
              jax.ShapeDtypeStruct((2, NP), jnp.float32)),
    mesh=_mesh,
    scratch_types=[
        pltpu.VMEM((B,), jnp.int32),
        pltpu.VMEM((B,), jnp.int32),
        pltpu.VMEM((B,), jnp.int32),
        pltpu.VMEM((B,), jnp.int32),
        pltpu.VMEM((B,), jnp.int32),
        pltpu.VMEM((B,), jnp.int32),
        pltpu.VMEM((B,), jnp.float32),
        pltpu.VMEM((B,), jnp.float32),
        pltpu.VMEM((B,), jnp.float32),
        pltpu.VMEM((B, DH), jnp.float32),
        pltpu.VMEM((B, DH), jnp.float32),
        pltpu.VMEM((B, DH), jnp.float32),
        pltpu.VMEM((ROWS_PER_TILE,), jnp.float32),
        pltpu.VMEM_SHARED((NP, DH), jnp.float32),
        pltpu.VMEM_SHARED((NP,), jnp.float32),
        pltpu.SemaphoreType.DMA,
        pltpu.SemaphoreType.DMA,
        pltpu.SemaphoreType.DMA,
        pltpu.SemaphoreType.DMA,
        pltpu.SemaphoreType.DMA,
        pltpu.SemaphoreType.DMA,
    ],
)(_sc_agg_body)


def _dense_body(h_ref, agg_ref, ws_ref, W_ref, b_ref, out_ref):
    hl = h_ref[0]
    hh = h_ref[1]
    inv = 1.0 / (ws_ref[0] + ws_ref[1] + 1e-9)
    al = agg_ref[0] * inv
    ah = agg_ref[1] * inv
    W = W_ref[...]
    z = (jnp.dot(hl, W[0:128, :], preferred_element_type=jnp.float32)
         + jnp.dot(hh, W[128:256, :], preferred_element_type=jnp.float32)
         + jnp.dot(al, W[256:384, :], preferred_element_type=jnp.float32)
         + jnp.dot(ah, W[384:512, :], preferred_element_type=jnp.float32)
         + b_ref[...])
    z = jnp.maximum(z, 0.0)
    z = z / (jnp.sqrt(jnp.sum(z * z, axis=1, keepdims=True)) + 1e-9)
    out_ref[0, :, :] = z[:, :DH]
    out_ref[1, :, :] = z[:, DH:]


_R = 256

_dense = pl.pallas_call(
    _dense_body,
    grid=(NP // _R,),
    in_specs=[
        pl.BlockSpec((2, _R, DH), lambda i: (0, i, 0)),
        pl.BlockSpec((2, _R, DH), lambda i: (0, i, 0)),
        pl.BlockSpec((2, _R, 1), lambda i: (0, i, 0)),
        pl.BlockSpec((2 * D, D), lambda i: (0, 0)),
        pl.BlockSpec((1, D), lambda i: (0, 0)),
    ],
    out_specs=pl.BlockSpec((2, _R, DH), lambda i: (0, i, 0)),
    out_shape=jax.ShapeDtypeStruct((2, NP, DH), jnp.float32),
)


def kernel(x, edge_index, edge_weight, W0, b0, W1, b1):
    h3 = jnp.pad(x, ((0, NP - N), (0, 0))).reshape(NP, 2, DH).transpose(1, 0, 2)
    for W, b in ((W0, b0), (W1, b1)):
        agg3, ws = _sc_agg(h3, edge_index[0], edge_index[1], edge_weight)
        h3 = _dense(h3, agg3, ws.reshape(2, NP, 1), W, b.reshape(1, D))
    return h3.transpose(1, 0, 2).reshape(NP, D)[:N]


# speedup vs baseline: 1.1405x; 1.1405x over previous
import functools

import jax
import jax.numpy as jnp
from jax import lax
from jax.experimental import pallas as pl
from jax.experimental.pallas import tpu as pltpu
from jax.experimental.pallas import tpu_sc as plsc

N = 10000
NP = 10240
E = 160000
D = 256
DH = 128
B = 128
NCHUNK = E // B
NTILES = 16
ROWS_PER_TILE = NP // NTILES
ZROWS = B

NCH_BASE = NCHUNK // NTILES
NCH_REM = NCHUNK % NTILES
NCH_CEIL = NCH_BASE + (2 if NCH_REM else 0)

_mesh = plsc.VectorSubcoreMesh(core_axis_name="c", subcore_axis_name="s")


def _sc_agg_body(x3, ei_h, w_h, agg3, ws_out,
                 idx2_a, idx2_b, w_a, w_b, rows_a, rows_b, zws_v,
                 acc_sh, ws_sh, sem_a, sem_b):
    c = lax.axis_index("c")
    s = lax.axis_index("s")

    def zrow(i, carry):
        for k in range(DH // 16):
            rows_a[i, k * 16:(k + 1) * 16] = jnp.zeros((16,), jnp.float32)
        return carry
    lax.fori_loop(0, ZROWS, zrow, 0)

    def zws_row(i, carry):
        zws_v[pl.ds(i * 16, 16)] = jnp.zeros((16,), jnp.float32)
        return carry
    lax.fori_loop(0, ROWS_PER_TILE // 16, zws_row, 0)

    base_rows = s * ROWS_PER_TILE
    for kk in range(ROWS_PER_TILE // ZROWS):
        pltpu.sync_copy(rows_a, acc_sh.at[pl.ds(base_rows + kk * ZROWS, ZROWS)])
    pltpu.sync_copy(zws_v, ws_sh.at[pl.ds(base_rows, ROWS_PER_TILE)])

    plsc.subcore_barrier()

    nch = NCH_BASE + jnp.where(s < NCH_REM, 1, 0)
    bufs = ((idx2_a, w_a, rows_a, sem_a), (idx2_b, w_b, rows_b, sem_b))

    def issue(jj, buf):
        idx2_v, w_v, rows_v, sem = buf
        base = (s + jj * NTILES) * B
        pltpu.sync_copy(ei_h.at[:, pl.ds(base, B)], idx2_v)
        pltpu.sync_copy(w_h.at[pl.ds(base, B)], w_v)
        pltpu.async_copy(x3.at[c].at[idx2_v.at[0]], rows_v, sem)

    issue(0, bufs[0])
    issue(1, bufs[1])

    @pl.loop(0, NCH_CEIL, step=2)
    def _chunks(j):
        for bsel in range(2):
            idx2_v, w_v, rows_v, sem = bufs[bsel]
            jj = j + bsel

            @pl.when(jj < nch)
            def _():
                pltpu.make_async_copy(
                    x3.at[c].at[idx2_v.at[0]], rows_v, sem).wait()

                def grp(g, rcarry):
                    wvec = w_v[pl.ds(g * 16, 16)]
                    for r in range(16):
                        wr = wvec[r]
                        row = g * 16 + r
                        for k in range(DH // 16):
                            sl = pl.ds(k * 16, 16)
                            rows_v[row, sl] = rows_v[row, sl] * wr
                    return rcarry
                lax.fori_loop(0, B // 16, grp, 0)

                pltpu.sync_copy(rows_v, acc_sh.at[idx2_v.at[1]], add=True)

                @pl.when((jj & 1) == c)
                def _():
                    pltpu.sync_copy(w_v, ws_sh.at[idx2_v.at[1]], add=True)

                @pl.when(jj + 2 < nch)
                def _():
                    issue(jj + 2, bufs[bsel])

    plsc.subcore_barrier()

    pltpu.sync_copy(acc_sh.at[pl.ds(base_rows, ROWS_PER_TILE)],
                    agg3.at[c].at[pl.ds(base_rows, ROWS_PER_TILE)])
    pltpu.sync_copy(ws_sh.at[pl.ds(base_rows, ROWS_PER_TILE)],
                    ws_out.at[c].at[pl.ds(base_rows, ROWS_PER_TILE)])


_sc_agg = functools.partial(
    pl.kernel,
    out_type=(jax.ShapeDtypeStruct((2, NP, DH), jnp.float32),
              jax.ShapeDtypeStruct((2, NP), jnp.float32)),
    mesh=_mesh,
    scratch_types=[
        pltpu.VMEM((2, B), jnp.int32),
        pltpu.VMEM((2, B), jnp.int32),
        pltpu.VMEM((B,), jnp.float32),
        pltpu.VMEM((B,), jnp.float32),
        pltpu.VMEM((B, DH), jnp.float32),
        pltpu.VMEM((B, DH), jnp.float32),
        pltpu.VMEM((ROWS_PER_TILE,), jnp.float32),
        pltpu.VMEM_SHARED((NP, DH), jnp.float32),
        pltpu.VMEM_SHARED((NP,), jnp.float32),
        pltpu.SemaphoreType.DMA,
        pltpu.SemaphoreType.DMA,
    ],
)(_sc_agg_body)


def _dense_body(h_ref, agg_ref, ws_ref, W_ref, b_ref, out_ref):
    hl = h_ref[0]
    hh = h_ref[1]
    inv = 1.0 / (ws_ref[0] + ws_ref[1] + 1e-9)
    al = agg_ref[0] * inv
    ah = agg_ref[1] * inv
    W = W_ref[...]
    z = (jnp.dot(hl, W[0:128, :], preferred_element_type=jnp.float32)
         + jnp.dot(hh, W[128:256, :], preferred_element_type=jnp.float32)
         + jnp.dot(al, W[256:384, :], preferred_element_type=jnp.float32)
         + jnp.dot(ah, W[384:512, :], preferred_element_type=jnp.float32)
         + b_ref[...])
    z = jnp.maximum(z, 0.0)
    z = z / (jnp.sqrt(jnp.sum(z * z, axis=1, keepdims=True)) + 1e-9)
    out_ref[0, :, :] = z[:, :DH]
    out_ref[1, :, :] = z[:, DH:]


_R = 256

_dense = pl.pallas_call(
    _dense_body,
    grid=(NP // _R,),
    in_specs=[
        pl.BlockSpec((2, _R, DH), lambda i: (0, i, 0)),
        pl.BlockSpec((2, _R, DH), lambda i: (0, i, 0)),
        pl.BlockSpec((2, _R, 1), lambda i: (0, i, 0)),
        pl.BlockSpec((2 * D, D), lambda i: (0, 0)),
        pl.BlockSpec((1, D), lambda i: (0, 0)),
    ],
    out_specs=pl.BlockSpec((2, _R, DH), lambda i: (0, i, 0)),
    out_shape=jax.ShapeDtypeStruct((2, NP, DH), jnp.float32),
)


def kernel(x, edge_index, edge_weight, W0, b0, W1, b1):
    h3 = jnp.pad(x, ((0, NP - N), (0, 0))).reshape(NP, 2, DH).transpose(1, 0, 2)
    for W, b in ((W0, b0), (W1, b1)):
        agg3, ws = _sc_agg(h3, edge_index, edge_weight)
        h3 = _dense(h3, agg3, ws.reshape(2, NP, 1), W, b.reshape(1, D))
    return h3.transpose(1, 0, 2).reshape(NP, D)[:N]
